# Initial kernel scaffold; baseline (speedup 1.0000x reference)
import jax
import jax.numpy as jnp
from jax.experimental import pallas as pl


def kernel(value_idx, turn_idx, value_table, turn_table, temporal_table):
    raise NotImplementedError("write your pallas kernel here")


# speedup vs baseline: 1.8128x; 1.8128x over previous
import functools

import jax
import jax.numpy as jnp
from jax import lax
from jax.experimental import pallas as pl
from jax.experimental.pallas import tpu as pltpu, tpu_sc as plsc

HIDDEN = 768
NVALS = 2000
NTURNS = 10
LANES = 16
NWORK = 32
CHUNK = 128


def _sc_lookup(n_rows):
    rpw = n_rows // NWORK
    nchunk = rpw // CHUNK
    mesh = plsc.VectorSubcoreMesh(core_axis_name="c", subcore_axis_name="s")

    @functools.partial(
        pl.kernel,
        out_type=jax.ShapeDtypeStruct((n_rows, HIDDEN), jnp.float32),
        mesh=mesh,
        scratch_types=[
            pltpu.VMEM((CHUNK,), jnp.int32),
            pltpu.VMEM((CHUNK,), jnp.int32),
            pltpu.VMEM((CHUNK, HIDDEN), jnp.float32),
            pltpu.VMEM((NTURNS * HIDDEN,), jnp.float32),
            pltpu.VMEM((NTURNS * HIDDEN,), jnp.float32),
            pltpu.VMEM((NTURNS * HIDDEN,), jnp.float32),
            pltpu.SemaphoreType.DMA,
        ],
    )
    def k(vidx_hbm, tidx_hbm, value_hbm, turn_hbm, temp_hbm, out_hbm,
          idx_v, tid_v, buf, comb_v, turn_v, temp_v, sem):
        wid = lax.axis_index("s") * 2 + lax.axis_index("c")

        pltpu.sync_copy(turn_hbm, turn_v)
        pltpu.sync_copy(temp_hbm, temp_v)

        def comb_body(i, carry):
            s = pl.ds(pl.multiple_of(i * LANES, LANES), LANES)
            comb_v[s] = turn_v[s] + temp_v[s]
            return carry
        lax.fori_loop(0, NTURNS * HIDDEN // LANES, comb_body, 0)

        def chunk_body(kk, carry):
            base = pl.multiple_of((wid * nchunk + kk) * CHUNK, CHUNK)
            pltpu.sync_copy(vidx_hbm.at[pl.ds(base, CHUNK)], idx_v)
            pltpu.sync_copy(tidx_hbm.at[pl.ds(base, CHUNK)], tid_v)
            pltpu.async_copy(value_hbm.at[idx_v], buf, sem).wait()

            def group_body(g, rcarry):
                tvec = tid_v[pl.ds(pl.multiple_of(g * LANES, LANES), LANES)]
                for i in range(LANES):
                    c = g * LANES + i
                    tb = tvec[i] * HIDDEN
                    for j in range(HIDDEN // LANES):
                        sl = pl.ds(j * LANES, LANES)
                        buf[c, sl] = buf[c, sl] + comb_v[pl.ds(tb + j * LANES, LANES)]
                return rcarry
            lax.fori_loop(0, CHUNK // LANES, group_body, 0)

            pltpu.sync_copy(buf, out_hbm.at[pl.ds(base, CHUNK)])
            return carry
        lax.fori_loop(0, nchunk, chunk_body, 0)

    return k


def kernel(value_idx, turn_idx, value_table, turn_table, temporal_table):
    b, l = value_idx.shape
    n = b * l
    vidx = value_idx.reshape(n).astype(jnp.int32)
    tidx = turn_idx.reshape(n).astype(jnp.int32)
    out = _sc_lookup(n)(
        vidx, tidx, value_table,
        turn_table.reshape(-1), temporal_table.reshape(-1),
    )
    return out.reshape(b, l, HIDDEN)
